# baseline (speedup 1.0000x reference)
import functools

import jax
import jax.numpy as jnp
from jax import lax
from jax.experimental import pallas as pl
from jax.experimental.pallas import tpu as pltpu
from jax.experimental.pallas import tpu_sc as plsc

EMBED_D = 32
B_TOTAL = 4096 * 50
NUM_CORES = 2
NUM_SUBCORES = 16
NW = NUM_CORES * NUM_SUBCORES
B_PER_W = B_TOTAL // NW
CHUNK = 1600
N_CHUNKS = B_PER_W // CHUNK

_mesh = plsc.VectorSubcoreMesh(core_axis_name="c", subcore_axis_name="s")


@functools.partial(
    pl.kernel,
    mesh=_mesh,
    out_type=jax.ShapeDtypeStruct((B_TOTAL, EMBED_D), jnp.float32),
    scratch_types=[
        pltpu.VMEM((CHUNK,), jnp.int32),
        pltpu.VMEM((CHUNK, EMBED_D), jnp.float32),
        pltpu.SemaphoreType.DMA,
    ],
    compiler_params=pltpu.CompilerParams(use_tc_tiling_on_sc=False),
)
def _gather_kernel(idx_hbm, table_hbm, out_hbm, idx_v, rows_v, sem):
    wid = lax.axis_index("s") * NUM_CORES + lax.axis_index("c")
    base = wid * B_PER_W
    for i in range(N_CHUNKS):
        off = base + i * CHUNK
        pltpu.sync_copy(idx_hbm.at[pl.ds(off, CHUNK)], idx_v)
        pltpu.async_copy(table_hbm.at[idx_v], rows_v, sem).wait()
        pltpu.sync_copy(rows_v, out_hbm.at[pl.ds(off, CHUNK)])


def kernel(x, wordmat):
    idx = x.reshape(-1).astype(jnp.int32)
    out = _gather_kernel(idx, wordmat)
    return out.reshape(x.shape + (EMBED_D,))


# speedup vs baseline: 1.0008x; 1.0008x over previous
import functools

import jax
import jax.numpy as jnp
from jax import lax
from jax.experimental import pallas as pl
from jax.experimental.pallas import tpu as pltpu
from jax.experimental.pallas import tpu_sc as plsc

EMBED_D = 32
B_TOTAL = 4096 * 50
NUM_CORES = 2
NUM_SUBCORES = 16
NW = NUM_CORES * NUM_SUBCORES
B_PER_W = B_TOTAL // NW
CHUNK = 1600
N_CHUNKS = B_PER_W // CHUNK

_mesh = plsc.VectorSubcoreMesh(core_axis_name="c", subcore_axis_name="s")


@functools.partial(
    pl.kernel,
    mesh=_mesh,
    out_type=jax.ShapeDtypeStruct((B_TOTAL, EMBED_D), jnp.float32),
    scratch_types=[
        pltpu.VMEM((B_PER_W,), jnp.int32),
        pltpu.VMEM((CHUNK, EMBED_D), jnp.float32),
        pltpu.VMEM((CHUNK, EMBED_D), jnp.float32),
        pltpu.SemaphoreType.DMA,
        pltpu.SemaphoreType.DMA,
        pltpu.SemaphoreType.DMA,
    ],
    compiler_params=pltpu.CompilerParams(use_tc_tiling_on_sc=False),
)
def _gather_kernel(idx_hbm, table_hbm, out_hbm, idx_v, rows_a, rows_b,
                   g_sem, wa_sem, wb_sem):
    wid = lax.axis_index("s") * NUM_CORES + lax.axis_index("c")
    base = wid * B_PER_W
    pltpu.sync_copy(idx_hbm.at[pl.ds(base, B_PER_W)], idx_v)

    bufs = (rows_a, rows_b)
    wsems = (wa_sem, wb_sem)

    def gather(i, buf):
        return pltpu.async_copy(
            table_hbm.at[idx_v.at[pl.ds(i * CHUNK, CHUNK)]], buf, g_sem)

    def put(i, buf, wsem):
        return pltpu.async_copy(
            buf, out_hbm.at[pl.ds(base + i * CHUNK, CHUNK)], wsem)

    writes = [None, None]
    gather(0, bufs[0]).wait()
    for i in range(N_CHUNKS):
        nxt = i + 1
        if nxt < N_CHUNKS:
            p = nxt % 2
            if writes[p] is not None:
                writes[p].wait()
                writes[p] = None
            g = gather(nxt, bufs[p])
        writes[i % 2] = put(i, bufs[i % 2], wsems[i % 2])
        if nxt < N_CHUNKS:
            g.wait()
    for w in writes:
        if w is not None:
            w.wait()


def kernel(x, wordmat):
    idx = x.reshape(-1).astype(jnp.int32)
    out = _gather_kernel(idx, wordmat)
    return out.reshape(x.shape + (EMBED_D,))
